# baseline (speedup 1.0000x reference)
import jax
import jax.numpy as jnp
from jax import lax
from jax.experimental import pallas as pl
from jax.experimental.pallas import tpu as pltpu
from jax.experimental.pallas import tpu_sc as plsc

VOCAB = 1000000
EMB_DIM = 32
BATCH = 16384
FIELDS = 26

_info = plsc.get_sparse_core_info()
NC, NS = _info.num_cores, _info.num_subcores
NW = NC * NS

BLK = 128
NBLK = BATCH // BLK
UNITS = FIELDS * NBLK
PER_W = UNITS // NW
LPR = VOCAB // 4

assert UNITS % NW == 0
assert PER_W % 2 == 0


def _body(ids_hbm, table_hbm, out_hbm,
          idx0, idx1, idx4_0, idx4_1, colb0, colb1,
          rows0, rows1, ot0, ot1, gsem0, gsem1, osem0, osem1):
    bufs = ((idx0, idx4_0, colb0, rows0, ot0, gsem0, osem0),
            (idx1, idx4_1, colb1, rows1, ot1, gsem1, osem1))
    wid = lax.axis_index("s") * NC + lax.axis_index("c")
    u0 = wid * PER_W

    def unit_fj(t):
        u = u0 + t
        f = u // NBLK
        j = u - f * NBLK
        return f, j

    def stage(t, b):
        idx, idx4, colb, rows, _, gsem, _ = bufs[b]
        f, j = unit_fj(t)
        pltpu.sync_copy(ids_hbm.at[f, pl.ds(j * BLK, BLK)], idx)
        for g in range(BLK // 16):
            x = idx[pl.ds(g * 16, 16)]
            idx4[pl.ds(g * 16, 16)] = lax.shift_right_logical(x, 2)
            colb[pl.ds(g * 16, 16)] = lax.shift_left(
                lax.bitwise_and(x, 3), 5)
        pltpu.async_copy(table_hbm.at[idx4], rows, gsem)

    def wait_gather(b):
        _, idx4, _, rows, _, gsem, _ = bufs[b]
        pltpu.make_async_copy(table_hbm.at[idx4], rows, gsem).wait()

    def start_scatter(t, b):
        ot, osem = bufs[b][4], bufs[b][6]
        f, j = unit_fj(t)
        pltpu.async_copy(ot, out_hbm.at[f, :, pl.ds(j * BLK, BLK)], osem)

    def wait_scatter(t, b):
        ot, osem = bufs[b][4], bufs[b][6]
        f, j = unit_fj(t)
        pltpu.make_async_copy(ot, out_hbm.at[f, :, pl.ds(j * BLK, BLK)],
                              osem).wait()

    def extract(b):
        _, _, colb_r, rows, ot, _, _ = bufs[b]
        iota = lax.iota(jnp.int32, 16)
        for g in range(BLK // 16):
            row_i = iota + (g * 16)
            colb = colb_r[pl.ds(g * 16, 16)]
            for e in range(EMB_DIM):
                val = plsc.load_gather(rows, [row_i, colb + e])
                ot[e, pl.ds(g * 16, 16)] = val

    stage(0, 0)
    stage(1, 1)
    wait_gather(0)
    extract(0)
    start_scatter(0, 0)
    stage(2, 0)
    wait_gather(1)
    extract(1)
    start_scatter(1, 1)
    stage(3, 1)

    def pair(h):
        t = 2 * h
        wait_gather(0)
        wait_scatter(t - 2, 0)
        extract(0)
        start_scatter(t, 0)
        stage(t + 2, 0)
        wait_gather(1)
        wait_scatter(t - 1, 1)
        extract(1)
        start_scatter(t + 1, 1)
        stage(t + 3, 1)

    pl.loop(1, PER_W // 2 - 1)(pair)

    t = PER_W - 2
    wait_gather(0)
    wait_scatter(t - 2, 0)
    extract(0)
    start_scatter(t, 0)
    wait_gather(1)
    wait_scatter(t - 1, 1)
    extract(1)
    start_scatter(t + 1, 1)
    wait_scatter(PER_W - 2, 0)
    wait_scatter(PER_W - 1, 1)


def kernel(ids, table):
    ids_t = jnp.swapaxes(ids.astype(jnp.int32), 0, 1)
    table4 = table.reshape(LPR, 128)

    mesh = plsc.VectorSubcoreMesh(core_axis_name="c", subcore_axis_name="s")
    out3 = pl.kernel(
        _body,
        out_type=jax.ShapeDtypeStruct((FIELDS, EMB_DIM, BATCH), jnp.float32),
        mesh=mesh,
        scratch_types=(
            [pltpu.VMEM((BLK,), jnp.int32)] * 6
            + [pltpu.VMEM((BLK, 128), jnp.float32)] * 2
            + [pltpu.VMEM((EMB_DIM, BLK), jnp.float32)] * 2
            + [pltpu.SemaphoreType.DMA] * 4
        ),
        compiler_params=pltpu.CompilerParams(use_tc_tiling_on_sc=False,
                                             needs_layout_passes=False),
    )(ids_t, table4)
    return jnp.transpose(out3, (2, 0, 1))
